# baseline (speedup 1.0000x reference)
import functools

import jax
import jax.numpy as jnp
from jax.experimental import pallas as pl
from jax.experimental.pallas import tpu as pltpu

B, T, D = 2, 4096, 2048
E = 64
TOP_K = 2
TILE = 512

_NEG_INF = float("-inf")


def _router_kernel(x_ref, w_ref, logits_ref, weights_ref, indices_ref):
    x = x_ref[...]
    w = w_ref[...]
    logits = jax.lax.dot_general(
        x, w, (((1,), (1,)), ((), ())), preferred_element_type=jnp.float32
    )
    logits_ref[...] = logits

    idx = jax.lax.broadcasted_iota(jnp.int32, logits.shape, 1)
    m1 = jnp.max(logits, axis=-1, keepdims=True)
    i1 = jnp.min(jnp.where(logits == m1, idx, E), axis=-1, keepdims=True)
    masked = jnp.where(idx == i1, _NEG_INF, logits)
    m2 = jnp.max(masked, axis=-1, keepdims=True)
    i2 = jnp.min(jnp.where(masked == m2, idx, E), axis=-1, keepdims=True)

    e2 = jnp.exp(m2 - m1)
    denom = 1.0 + e2
    w1 = 1.0 / denom
    w2 = e2 / denom

    weights_ref[...] = jnp.concatenate([w1, w2], axis=-1)
    indices_ref[...] = jnp.concatenate([i1, i2], axis=-1)


@jax.jit
def kernel(x, W):
    xt = x.reshape(B * T, D)
    grid = (B * T) // TILE
    logits, weights, indices = pl.pallas_call(
        _router_kernel,
        grid=(grid,),
        in_specs=[
            pl.BlockSpec((TILE, D), lambda i: (i, 0)),
            pl.BlockSpec((E, D), lambda i: (0, 0)),
        ],
        out_specs=[
            pl.BlockSpec((TILE, E), lambda i: (i, 0)),
            pl.BlockSpec((TILE, TOP_K), lambda i: (i, 0)),
            pl.BlockSpec((TILE, TOP_K), lambda i: (i, 0)),
        ],
        out_shape=[
            jax.ShapeDtypeStruct((B * T, E), jnp.float32),
            jax.ShapeDtypeStruct((B * T, TOP_K), jnp.float32),
            jax.ShapeDtypeStruct((B * T, TOP_K), jnp.int32),
        ],
        compiler_params=pltpu.CompilerParams(
            dimension_semantics=("parallel",),
        ),
    )(xt, W)
    return (
        weights.reshape(B, T, TOP_K),
        indices.reshape(B, T, TOP_K),
        logits.reshape(B, T, E),
    )


# speedup vs baseline: 1.1223x; 1.1223x over previous
import functools

import jax
import jax.numpy as jnp
from jax.experimental import pallas as pl
from jax.experimental.pallas import tpu as pltpu

B, T, D = 2, 4096, 2048
E = 64
TOP_K = 2
TILE = 1024

_NEG_INF = float("-inf")


def _router_kernel(x_ref, w_ref, logits_ref, weights_ref, indices_ref):
    x = x_ref[...]
    w = w_ref[...]
    logits = jax.lax.dot_general(
        x, w, (((1,), (1,)), ((), ())), preferred_element_type=jnp.float32
    )
    logits_ref[...] = logits

    idx = jax.lax.broadcasted_iota(jnp.int32, logits.shape, 1)
    m1 = jnp.max(logits, axis=-1, keepdims=True)
    i1 = jnp.min(jnp.where(logits == m1, idx, E), axis=-1, keepdims=True)
    masked = jnp.where(idx == i1, _NEG_INF, logits)
    m2 = jnp.max(masked, axis=-1, keepdims=True)
    i2 = jnp.min(jnp.where(masked == m2, idx, E), axis=-1, keepdims=True)

    e2 = jnp.exp(m2 - m1)
    denom = 1.0 + e2
    w1 = 1.0 / denom
    w2 = e2 / denom

    weights_ref[...] = jnp.concatenate([w1, w2], axis=-1)
    indices_ref[...] = jnp.concatenate([i1, i2], axis=-1)


@jax.jit
def kernel(x, W):
    xt = x.reshape(B * T, D)
    grid = (B * T) // TILE
    logits, weights, indices = pl.pallas_call(
        _router_kernel,
        grid=(grid,),
        in_specs=[
            pl.BlockSpec((TILE, D), lambda i: (i, 0)),
            pl.BlockSpec((E, D), lambda i: (0, 0)),
        ],
        out_specs=[
            pl.BlockSpec((TILE, E), lambda i: (i, 0)),
            pl.BlockSpec((TILE, TOP_K), lambda i: (i, 0)),
            pl.BlockSpec((TILE, TOP_K), lambda i: (i, 0)),
        ],
        out_shape=[
            jax.ShapeDtypeStruct((B * T, E), jnp.float32),
            jax.ShapeDtypeStruct((B * T, TOP_K), jnp.float32),
            jax.ShapeDtypeStruct((B * T, TOP_K), jnp.int32),
        ],
        compiler_params=pltpu.CompilerParams(
            dimension_semantics=("parallel",),
        ),
    )(xt, W)
    return (
        weights.reshape(B, T, TOP_K),
        indices.reshape(B, T, TOP_K),
        logits.reshape(B, T, E),
    )
